# baseline (speedup 1.0000x reference)
import functools

import jax
import jax.numpy as jnp
from jax import lax
from jax.experimental import pallas as pl
from jax.experimental.pallas import tpu as pltpu
from jax.experimental.pallas import tpu_sc as plsc

HID = 128
EPS = 1e-12
CHUNK = 128


@functools.lru_cache(maxsize=None)
def _make_sc_gather(n_tokens: int, width: int):
    info = plsc.get_sparse_core_info()
    nc, ns = info.num_cores, info.num_subcores
    nw = nc * ns
    per_w = n_tokens // nw
    iters = per_w // CHUNK
    mesh = plsc.VectorSubcoreMesh(core_axis_name="c", subcore_axis_name="s")

    npairs = iters // 2

    @functools.partial(
        pl.kernel,
        out_type=jax.ShapeDtypeStruct((n_tokens, width), jnp.int32),
        mesh=mesh,
        scratch_types=[
            pltpu.VMEM((per_w,), jnp.int32),
            pltpu.VMEM((CHUNK, width), jnp.int32),
            pltpu.VMEM((CHUNK, width), jnp.int32),
            pltpu.SemaphoreType.DMA,
            pltpu.SemaphoreType.DMA,
        ],
        compiler_params=pltpu.CompilerParams(use_tc_tiling_on_sc=False),
    )
    def gather(table_hbm, ids_hbm, out_hbm, idx_v, rows0, rows1, sem0, sem1):
        wid = lax.axis_index("s") * nc + lax.axis_index("c")
        base = wid * per_w
        pltpu.sync_copy(ids_hbm.at[pl.ds(base, per_w)], idx_v)

        def g(i, rows, sem):
            return pltpu.make_async_copy(
                table_hbm.at[idx_v.at[pl.ds(i * CHUNK, CHUNK)]], rows, sem)

        g(0, rows0, sem0).start()

        def body(j, carry):
            i0 = 2 * j
            g(i0 + 1, rows1, sem1).start()
            g(i0, rows0, sem0).wait()
            pltpu.sync_copy(rows0, out_hbm.at[pl.ds(base + i0 * CHUNK, CHUNK)])

            @pl.when(j < npairs - 1)
            def _():
                g(i0 + 2, rows0, sem0).start()

            g(i0 + 1, rows1, sem1).wait()
            pltpu.sync_copy(
                rows1, out_hbm.at[pl.ds(base + (i0 + 1) * CHUNK, CHUNK)])
            return carry

        lax.fori_loop(0, npairs, body, 0)

    return gather


def _ln_body(gath_ref, code_ref, pos_ref, ctab_ref, gamma_ref, beta_ref,
             out_ref):
    r, s = gath_ref.shape[0], gath_ref.shape[1]
    j = jnp.full((HID, HID), 1.0 / HID, dtype=jnp.float32)
    for q in range(r):
        code = code_ref[q, 0, :][:, None]
        oh = (code == lax.broadcasted_iota(jnp.int32, (s, 32), 1)
              ).astype(jnp.float32)
        e = (gath_ref[q].astype(jnp.float32) + pos_ref[...]
             + jnp.dot(oh, ctab_ref[...], preferred_element_type=jnp.float32))
        m1 = jnp.dot(e, j, preferred_element_type=jnp.float32)
        m2 = jnp.dot(e * e, j, preferred_element_type=jnp.float32)
        out_ref[q] = ((e - m1) * lax.rsqrt(m2 - m1 * m1 + EPS)
                      * gamma_ref[...] + beta_ref[...])


def _ln_body_chain(dst_ref, gath_ref, code_ref, pos_ref, ctab_ref, gamma_ref,
                   beta_ref, out_ref):
    del dst_ref
    _ln_body(gath_ref, code_ref, pos_ref, ctab_ref, gamma_ref, beta_ref,
             out_ref)


def kernel(input_ids, token_type_ids, task_type_ids, word_emb, pos_emb,
           tok_emb, task_emb, gamma, beta):
    b, s = input_ids.shape
    nsl = 4
    rows = 16
    bsl = b // nsl
    grid_k = bsl // rows
    ids = input_ids.reshape(nsl, bsl * s).astype(jnp.int32)
    code4 = (token_type_ids.astype(jnp.int32)
             + 2 * task_type_ids.astype(jnp.int32)).reshape(nsl, bsl, 1, s)
    ar = jnp.arange(32)
    ctab = tok_emb[ar % 2] + task_emb[ar // 2]
    gamma2 = gamma.reshape(1, HID)
    beta2 = beta.reshape(1, HID)

    vocab = word_emb.shape[0]
    word32 = lax.bitcast_convert_type(
        word_emb.astype(jnp.bfloat16).reshape(vocab, HID // 2, 2), jnp.int32)
    sc_gather = _make_sc_gather(bsl * s, HID // 2)
    gaths = [
        lax.bitcast_convert_type(sc_gather(word32, ids[k]),
                                 jnp.bfloat16).reshape(bsl, s, HID)
        for k in range(nsl)
    ]

    in_specs = [
        pl.BlockSpec((rows, s, HID), lambda i: (i, 0, 0)),
        pl.BlockSpec((rows, 1, s), lambda i: (i, 0, 0)),
        pl.BlockSpec((s, HID), lambda i: (0, 0)),
        pl.BlockSpec((32, HID), lambda i: (0, 0)),
        pl.BlockSpec((1, HID), lambda i: (0, 0)),
        pl.BlockSpec((1, HID), lambda i: (0, 0)),
    ]
    out_shape = jax.ShapeDtypeStruct((b, s, HID), jnp.float32)
    cparams = pltpu.CompilerParams(dimension_semantics=("arbitrary",))

    out = None
    for k in range(nsl):
        args = (gaths[k], code4[k], pos_emb, ctab, gamma2, beta2)
        out_spec = pl.BlockSpec(
            (rows, s, HID), lambda i, kk=k: (kk * grid_k + i, 0, 0))
        if k == 0:
            out = pl.pallas_call(
                _ln_body, grid=(grid_k,), in_specs=in_specs,
                out_specs=out_spec, out_shape=out_shape,
                compiler_params=cparams)(*args)
        else:
            out = pl.pallas_call(
                _ln_body_chain, grid=(grid_k,),
                in_specs=[pl.BlockSpec(memory_space=pl.ANY)] + in_specs,
                out_specs=out_spec, out_shape=out_shape,
                input_output_aliases={0: 0},
                compiler_params=cparams)(out, *args)
    return out


# speedup vs baseline: 4.9348x; 4.9348x over previous
import functools

import jax
import jax.numpy as jnp
from jax import lax
from jax.experimental import pallas as pl
from jax.experimental.pallas import tpu as pltpu
from jax.experimental.pallas import tpu_sc as plsc

HID = 128
EPS = 1e-12
CHUNK = 128


@functools.lru_cache(maxsize=None)
def _make_sc_gather(n_tokens: int):
    info = plsc.get_sparse_core_info()
    nc, ns = info.num_cores, info.num_subcores
    nw = nc * ns
    per_w = n_tokens // nw
    iters = per_w // CHUNK
    mesh = plsc.VectorSubcoreMesh(core_axis_name="c", subcore_axis_name="s")

    npairs = iters // 2

    @functools.partial(
        pl.kernel,
        out_type=jax.ShapeDtypeStruct((n_tokens, HID), jnp.float32),
        mesh=mesh,
        scratch_types=[
            pltpu.VMEM((per_w,), jnp.int32),
            pltpu.VMEM((CHUNK, HID), jnp.float32),
            pltpu.VMEM((CHUNK, HID), jnp.float32),
            pltpu.SemaphoreType.DMA,
            pltpu.SemaphoreType.DMA,
        ],
    )
    def gather(table_hbm, ids_hbm, out_hbm, idx_v, rows0, rows1, sem0, sem1):
        wid = lax.axis_index("s") * nc + lax.axis_index("c")
        base = wid * per_w
        pltpu.sync_copy(ids_hbm.at[pl.ds(base, per_w)], idx_v)

        def g(i, rows, sem):
            return pltpu.make_async_copy(
                table_hbm.at[idx_v.at[pl.ds(i * CHUNK, CHUNK)]], rows, sem)

        g(0, rows0, sem0).start()

        def body(j, carry):
            i0 = 2 * j
            g(i0 + 1, rows1, sem1).start()
            g(i0, rows0, sem0).wait()
            pltpu.sync_copy(rows0, out_hbm.at[pl.ds(base + i0 * CHUNK, CHUNK)])

            @pl.when(j < npairs - 1)
            def _():
                g(i0 + 2, rows0, sem0).start()

            g(i0 + 1, rows1, sem1).wait()
            pltpu.sync_copy(
                rows1, out_hbm.at[pl.ds(base + (i0 + 1) * CHUNK, CHUNK)])
            return carry

        lax.fori_loop(0, npairs, body, 0)

    return gather


def _ln_body(gath_ref, code_ref, pos_ref, ctab_ref, gamma_ref, beta_ref,
             out_ref):
    r, s = gath_ref.shape[0], gath_ref.shape[1]
    j = jnp.full((HID, HID), 1.0 / HID, dtype=jnp.float32)
    for q in range(r):
        code = code_ref[q, 0, :][:, None]
        oh = (code == lax.broadcasted_iota(jnp.int32, (s, 32), 1)
              ).astype(jnp.float32)
        e = (gath_ref[q] + pos_ref[...]
             + jnp.dot(oh, ctab_ref[...], preferred_element_type=jnp.float32))
        m1 = jnp.dot(e, j, preferred_element_type=jnp.float32)
        m2 = jnp.dot(e * e, j, preferred_element_type=jnp.float32)
        out_ref[q] = ((e - m1) * lax.rsqrt(m2 - m1 * m1 + EPS)
                      * gamma_ref[...] + beta_ref[...])


def _ln_body_chain(dst_ref, gath_ref, code_ref, pos_ref, ctab_ref, gamma_ref,
                   beta_ref, out_ref):
    del dst_ref
    _ln_body(gath_ref, code_ref, pos_ref, ctab_ref, gamma_ref, beta_ref,
             out_ref)


def kernel(input_ids, token_type_ids, task_type_ids, word_emb, pos_emb,
           tok_emb, task_emb, gamma, beta):
    b, s = input_ids.shape
    nsl = 8
    rows = 16
    bsl = b // nsl
    grid_k = bsl // rows
    ids = input_ids.reshape(nsl, bsl * s).astype(jnp.int32)
    code4 = (token_type_ids.astype(jnp.int32)
             + 2 * task_type_ids.astype(jnp.int32)).reshape(nsl, bsl, 1, s)
    ar = jnp.arange(32)
    ctab = tok_emb[ar % 2] + task_emb[ar // 2]
    gamma2 = gamma.reshape(1, HID)
    beta2 = beta.reshape(1, HID)

    sc_gather = _make_sc_gather(bsl * s)
    gaths = [sc_gather(word_emb, ids[k]).reshape(bsl, s, HID)
             for k in range(nsl)]

    in_specs = [
        pl.BlockSpec((rows, s, HID), lambda i: (i, 0, 0)),
        pl.BlockSpec((rows, 1, s), lambda i: (i, 0, 0)),
        pl.BlockSpec((s, HID), lambda i: (0, 0)),
        pl.BlockSpec((32, HID), lambda i: (0, 0)),
        pl.BlockSpec((1, HID), lambda i: (0, 0)),
        pl.BlockSpec((1, HID), lambda i: (0, 0)),
    ]
    out_shape = jax.ShapeDtypeStruct((b, s, HID), jnp.float32)
    cparams = pltpu.CompilerParams(dimension_semantics=("arbitrary",))

    out = None
    for k in range(nsl):
        args = (gaths[k], code4[k], pos_emb, ctab, gamma2, beta2)
        out_spec = pl.BlockSpec(
            (rows, s, HID), lambda i, kk=k: (kk * grid_k + i, 0, 0))
        if k == 0:
            out = pl.pallas_call(
                _ln_body, grid=(grid_k,), in_specs=in_specs,
                out_specs=out_spec, out_shape=out_shape,
                compiler_params=cparams)(*args)
        else:
            out = pl.pallas_call(
                _ln_body_chain, grid=(grid_k,),
                in_specs=[pl.BlockSpec(memory_space=pl.ANY)] + in_specs,
                out_specs=out_spec, out_shape=out_shape,
                input_output_aliases={0: 0},
                compiler_params=cparams)(out, *args)
    return out


# speedup vs baseline: 5.1027x; 1.0340x over previous
import functools

import jax
import jax.numpy as jnp
from jax import lax
from jax.experimental import pallas as pl
from jax.experimental.pallas import tpu as pltpu
from jax.experimental.pallas import tpu_sc as plsc

HID = 128
EPS = 1e-12
CHUNK = 128


@functools.lru_cache(maxsize=None)
def _make_sc_gather(n_tokens: int):
    info = plsc.get_sparse_core_info()
    nc, ns = info.num_cores, info.num_subcores
    nw = nc * ns
    per_w = n_tokens // nw
    iters = per_w // CHUNK
    mesh = plsc.VectorSubcoreMesh(core_axis_name="c", subcore_axis_name="s")

    npairs = iters // 2

    @functools.partial(
        pl.kernel,
        out_type=jax.ShapeDtypeStruct((n_tokens, HID), jnp.float32),
        mesh=mesh,
        scratch_types=[
            pltpu.VMEM((per_w,), jnp.int32),
            pltpu.VMEM((CHUNK, HID), jnp.float32),
            pltpu.VMEM((CHUNK, HID), jnp.float32),
            pltpu.SemaphoreType.DMA,
            pltpu.SemaphoreType.DMA,
        ],
    )
    def gather(table_hbm, ids_hbm, out_hbm, idx_v, rows0, rows1, sem0, sem1):
        wid = lax.axis_index("s") * nc + lax.axis_index("c")
        base = wid * per_w
        pltpu.sync_copy(ids_hbm.at[pl.ds(base, per_w)], idx_v)

        def g(i, rows, sem):
            return pltpu.make_async_copy(
                table_hbm.at[idx_v.at[pl.ds(i * CHUNK, CHUNK)]], rows, sem)

        g(0, rows0, sem0).start()

        def body(j, carry):
            i0 = 2 * j
            g(i0 + 1, rows1, sem1).start()
            g(i0, rows0, sem0).wait()
            pltpu.sync_copy(rows0, out_hbm.at[pl.ds(base + i0 * CHUNK, CHUNK)])

            @pl.when(j < npairs - 1)
            def _():
                g(i0 + 2, rows0, sem0).start()

            g(i0 + 1, rows1, sem1).wait()
            pltpu.sync_copy(
                rows1, out_hbm.at[pl.ds(base + (i0 + 1) * CHUNK, CHUNK)])
            return carry

        lax.fori_loop(0, npairs, body, 0)

    return gather


def _ln_body(gath_ref, code_ref, pos_ref, ctab_ref, gamma_ref, beta_ref,
             out_ref):
    r, s = gath_ref.shape[0], gath_ref.shape[1]
    j = jnp.full((HID, HID), 1.0 / HID, dtype=jnp.float32)
    for q in range(r):
        code = code_ref[q, 0, :][:, None]
        oh = (code == lax.broadcasted_iota(jnp.int32, (s, 32), 1)
              ).astype(jnp.float32)
        e = (gath_ref[q] + pos_ref[...]
             + jnp.dot(oh, ctab_ref[...], preferred_element_type=jnp.float32))
        m1 = jnp.dot(e, j, preferred_element_type=jnp.float32)
        m2 = jnp.dot(e * e, j, preferred_element_type=jnp.float32)
        out_ref[q] = ((e - m1) * lax.rsqrt(m2 - m1 * m1 + EPS)
                      * gamma_ref[...] + beta_ref[...])


def _ln_body_chain(dst_ref, gath_ref, code_ref, pos_ref, ctab_ref, gamma_ref,
                   beta_ref, out_ref):
    del dst_ref
    _ln_body(gath_ref, code_ref, pos_ref, ctab_ref, gamma_ref, beta_ref,
             out_ref)


def kernel(input_ids, token_type_ids, task_type_ids, word_emb, pos_emb,
           tok_emb, task_emb, gamma, beta):
    b, s = input_ids.shape
    nsl = 4
    rows = 32
    bsl = b // nsl
    grid_k = bsl // rows
    ids = input_ids.reshape(nsl, bsl * s).astype(jnp.int32)
    code4 = (token_type_ids.astype(jnp.int32)
             + 2 * task_type_ids.astype(jnp.int32)).reshape(nsl, bsl, 1, s)
    ar = jnp.arange(32)
    ctab = tok_emb[ar % 2] + task_emb[ar // 2]
    gamma2 = gamma.reshape(1, HID)
    beta2 = beta.reshape(1, HID)

    sc_gather = _make_sc_gather(bsl * s)
    gaths = [sc_gather(word_emb, ids[k]).reshape(bsl, s, HID)
             for k in range(nsl)]

    in_specs = [
        pl.BlockSpec((rows, s, HID), lambda i: (i, 0, 0)),
        pl.BlockSpec((rows, 1, s), lambda i: (i, 0, 0)),
        pl.BlockSpec((s, HID), lambda i: (0, 0)),
        pl.BlockSpec((32, HID), lambda i: (0, 0)),
        pl.BlockSpec((1, HID), lambda i: (0, 0)),
        pl.BlockSpec((1, HID), lambda i: (0, 0)),
    ]
    out_shape = jax.ShapeDtypeStruct((b, s, HID), jnp.float32)
    cparams = pltpu.CompilerParams(dimension_semantics=("arbitrary",))

    out = None
    for k in range(nsl):
        args = (gaths[k], code4[k], pos_emb, ctab, gamma2, beta2)
        out_spec = pl.BlockSpec(
            (rows, s, HID), lambda i, kk=k: (kk * grid_k + i, 0, 0))
        if k == 0:
            out = pl.pallas_call(
                _ln_body, grid=(grid_k,), in_specs=in_specs,
                out_specs=out_spec, out_shape=out_shape,
                compiler_params=cparams)(*args)
        else:
            out = pl.pallas_call(
                _ln_body_chain, grid=(grid_k,),
                in_specs=[pl.BlockSpec(memory_space=pl.ANY)] + in_specs,
                out_specs=out_spec, out_shape=out_shape,
                input_output_aliases={0: 0},
                compiler_params=cparams)(out, *args)
    return out
